# baseline (speedup 1.0000x reference)
import functools

import jax
import jax.numpy as jnp
from jax.experimental import pallas as pl
from jax.experimental.pallas import tpu as pltpu


def _twohot_body(v_ref, b_ref, u1_ref, au1_ref, u2_ref, au2_ref, o_ref):
    v = jnp.maximum(v_ref[...], b_ref[0, 0, 0])[:, :, None]
    t1 = jnp.clip(v * u1_ref[...] - au1_ref[...], 0.0, 1.0)
    t2 = jnp.clip(v * u2_ref[...] - au2_ref[...], 0.0, 1.0)
    o_ref[...] = t1 - t2


def kernel(values, bin_values):
    r0, r1 = values.shape
    nbins = bin_values.shape[0]
    bins = bin_values
    u1i = 1.0 / (bins[1:] - bins[:-1])
    u1 = jnp.concatenate([jnp.zeros((1,), jnp.float32), u1i])
    au1 = jnp.concatenate([jnp.full((1,), -1.0, jnp.float32), bins[:-1] * u1i])
    nxt = jnp.concatenate([bins[1:], bins[-1:]])
    d2 = nxt - bins
    u2 = jnp.where(d2 > 0, 1.0 / jnp.maximum(d2, 1e-30), 0.0)
    au2 = bins * u2

    B = 512
    assert r0 % B == 0
    gi = r0 // B
    vt = values.T

    def c3(x):
        return x.reshape(1, 1, nbins)

    cspec = pl.BlockSpec((1, 1, nbins), lambda i: (0, 0, 0))
    out = pl.pallas_call(
        _twohot_body,
        grid=(gi,),
        in_specs=[
            pl.BlockSpec((r1, B), lambda i: (0, i)),
            cspec, cspec, cspec, cspec, cspec,
        ],
        out_specs=pl.BlockSpec((r1, B, nbins), lambda i: (0, i, 0)),
        out_shape=jax.ShapeDtypeStruct((r1, r0, nbins), jnp.float32),
        compiler_params=pltpu.CompilerParams(
            dimension_semantics=("arbitrary",),
        ),
    )(vt, c3(bins), c3(u1), c3(au1), c3(u2), c3(au2))
    return out.transpose(1, 0, 2)
